# Initial kernel scaffold; baseline (speedup 1.0000x reference)
import jax
import jax.numpy as jnp
from jax.experimental import pallas as pl


def kernel(x, W0, W1, W2, W3, W4, W5, W6, W7, W8):
    raise NotImplementedError("write your pallas kernel here")


# speedup vs baseline: 5.2429x; 5.2429x over previous
import functools

import jax
import jax.numpy as jnp
from jax import lax
from jax.experimental import pallas as pl
from jax.experimental.pallas import tpu as pltpu
from jax.experimental.pallas import tpu_sc as plsc

N = 100000
D = 128
L = 16
NC, NS = 2, 16
NW = NC * NS

C = 128
ROWS_PER_TILE = 3200
N_PAD = NW * ROWS_PER_TILE
CHUNKS = ROWS_PER_TILE // C

VA = (119, 10, 11)
VB = (12, 9, 5, 8, 2, 2)
ROWS_A = 119 * 10 * 11
ROWS_B = 12 * 9 * 5 * 8 * 2 * 2
T_ROWS = ROWS_A + ROWS_B

MULT_A = (10 * 11, 11, 1)
MULT_B = (9 * 5 * 8 * 2 * 2, 5 * 8 * 2 * 2, 8 * 2 * 2, 2 * 2, 2, 1)


def _body(xt_hbm, tbl_hbm, out_hbm, xcols, idx_a, idx_b, rows_a, rows_b,
          gsem):
    wid = lax.axis_index("s") * NC + lax.axis_index("c")
    base = wid * ROWS_PER_TILE

    def stage_indices(ci, buf):
        cbase = base + ci * C
        for f in range(9):
            pltpu.sync_copy(xt_hbm.at[pl.ds(f * N_PAD + cbase, C)],
                            xcols.at[f])

        def vec_body(j, _):
            s = pl.ds(j * L, L)
            ga = xcols[0, s] * MULT_A[0]
            for k in range(1, 3):
                ga = ga + xcols[k, s] * MULT_A[k]
            idx_a[buf, s] = ga
            gb = xcols[3, s] * MULT_B[0]
            for k in range(1, 6):
                gb = gb + xcols[3 + k, s] * MULT_B[k]
            idx_b[buf, s] = gb + ROWS_A
            return 0

        lax.fori_loop(0, C // L, vec_body, 0, unroll=True)

    def start_gathers(buf):
        pltpu.async_copy(tbl_hbm.at[idx_a.at[buf]], rows_a.at[buf], gsem)
        pltpu.async_copy(tbl_hbm.at[idx_b.at[buf]], rows_b.at[buf], gsem)

    def drain_gathers(buf):
        pltpu.make_async_copy(tbl_hbm.at[idx_a.at[buf]], rows_a.at[buf],
                              gsem).wait()
        pltpu.make_async_copy(tbl_hbm.at[idx_b.at[buf]], rows_b.at[buf],
                              gsem).wait()

    def sum_and_store(ci, buf):
        cbase = base + ci * C

        def row_body(r, _):
            for j in range(D // L):
                s = pl.ds(j * L, L)
                plsc.addupdate(rows_a.at[buf, r, s], rows_b[buf, r, s])
            return 0

        lax.fori_loop(0, C, row_body, 0)

        @pl.when(cbase + C <= N)
        def _():
            pltpu.sync_copy(rows_a.at[buf], out_hbm.at[pl.ds(cbase, C)])

        @pl.when(cbase + C > N)
        def _():
            @pl.when(cbase < N)
            def _():
                part = N % C
                pltpu.sync_copy(rows_a.at[buf, pl.ds(0, part)],
                                out_hbm.at[pl.ds(cbase, part)])

    stage_indices(0, 0)
    start_gathers(0)

    def chunk_body(ci, _):
        buf = lax.rem(ci, 2)
        nbuf = lax.rem(ci + 1, 2)

        @pl.when(ci + 1 < CHUNKS)
        def _():
            stage_indices(ci + 1, nbuf)
            start_gathers(nbuf)

        drain_gathers(buf)
        sum_and_store(ci, buf)
        return 0

    lax.fori_loop(0, CHUNKS, chunk_body, 0)


@jax.jit
def _encode(xt_pad, tbl):
    mesh = plsc.VectorSubcoreMesh(core_axis_name="c", subcore_axis_name="s",
                                  num_cores=NC, num_subcores=NS)
    f = pl.kernel(
        _body,
        out_type=jax.ShapeDtypeStruct((N, D), jnp.float32),
        mesh=mesh,
        scratch_types=[
            pltpu.VMEM((9, C), jnp.int32),
            pltpu.VMEM((2, C), jnp.int32),
            pltpu.VMEM((2, C), jnp.int32),
            pltpu.VMEM((2, C, D), jnp.float32),
            pltpu.VMEM((2, C, D), jnp.float32),
            pltpu.SemaphoreType.DMA,
        ],
    )
    return f(xt_pad, tbl)


def kernel(x, W0, W1, W2, W3, W4, W5, W6, W7, W8):
    ta = (W0[:, None, None, :] + W1[None, :, None, :] + W2[None, None, :, :])
    tb = (W3[:, None, None, None, None, None, :]
          + W4[None, :, None, None, None, None, :]
          + W5[None, None, :, None, None, None, :]
          + W6[None, None, None, :, None, None, :]
          + W7[None, None, None, None, :, None, :]
          + W8[None, None, None, None, None, :, :])
    tbl = jnp.concatenate(
        [ta.reshape(ROWS_A, D), tb.reshape(ROWS_B, D)], axis=0)
    xt = jnp.transpose(x).astype(jnp.int32)
    xt_pad = jnp.pad(xt, ((0, 0), (0, N_PAD - N))).reshape(9 * N_PAD)
    return _encode(xt_pad, tbl)
